# Initial kernel scaffold; baseline (speedup 1.0000x reference)
import jax
import jax.numpy as jnp
from jax.experimental import pallas as pl


def kernel(choices, rewards, alpha_plus, alpha_minus, initial_values):
    raise NotImplementedError("write your pallas kernel here")


# speedup vs baseline: 1.6336x; 1.6336x over previous
import jax
import jax.numpy as jnp
from jax.experimental import pallas as pl
from jax.experimental.pallas import tpu as pltpu

_K = 32


_TB = 8


def _body(sc_ref, ch_ref, rw_ref, out_ref, v_ref):
    t = pl.program_id(1)
    iv = sc_ref[0]
    ap = sc_ref[1]
    am = sc_ref[2]
    B = ch_ref.shape[0]

    @pl.when(t == 0)
    def _():
        v_ref[...] = jnp.full((B, _K), iv, jnp.float32)

    iota_k = jax.lax.broadcasted_iota(jnp.int32, (B, _K), 1)
    v = v_ref[...]
    for j in range(_TB):
        out_ref[:, j, :] = v
        ch = ch_ref[:, j, 0]
        rw = rw_ref[:, j, 0]
        onehot = iota_k == ch[:, None]
        chosen = jnp.sum(jnp.where(onehot, v, 0.0), axis=1)
        pe = rw - chosen
        pe = jnp.where(jnp.isnan(rw), 0.0, pe)
        coef = jnp.where(pe >= 0, ap, am)
        upd = chosen + coef * pe
        v = jnp.where(onehot, upd[:, None], v)
    v_ref[...] = v


def kernel(choices, rewards, alpha_plus, alpha_minus, initial_values):
    N, T = choices.shape
    iv = 100.0 * jnp.tanh(initial_values)
    ap = jax.nn.sigmoid(alpha_plus)
    am = jax.nn.sigmoid(alpha_minus)
    scalars = jnp.stack([iv, ap, am]).astype(jnp.float32)

    B = 1024
    grid = (N // B, T // _TB)
    ch3 = choices.reshape(N, T, 1)
    rw3 = rewards.reshape(N, T, 1)
    return pl.pallas_call(
        _body,
        grid=grid,
        in_specs=[
            pl.BlockSpec(memory_space=pltpu.SMEM),
            pl.BlockSpec((B, _TB, 1), lambda i, t: (i, t, 0)),
            pl.BlockSpec((B, _TB, 1), lambda i, t: (i, t, 0)),
        ],
        out_specs=pl.BlockSpec((B, _TB, _K), lambda i, t: (i, t, 0)),
        out_shape=jax.ShapeDtypeStruct((N, T, _K), jnp.float32),
        scratch_shapes=[pltpu.VMEM((B, _K), jnp.float32)],
    )(scalars, ch3, rw3)


# speedup vs baseline: 8.6689x; 5.3065x over previous
import functools

import jax
import jax.numpy as jnp
from jax import lax
from jax.experimental import pallas as pl
from jax.experimental.pallas import tpu as pltpu
from jax.experimental.pallas import tpu_sc as plsc

_K = 32
_L = 16
_NW = 32


def _sc_body(T, Tp, params_hbm, ch_hbm, rw_hbm, out_hbm,
             params_v, ch_v, rw_v, stage_v, sem):
    wid = lax.axis_index("s") * 2 + lax.axis_index("c")
    rows_per_w = ch_hbm.shape[0] // _NW
    groups = rows_per_w // _L

    pltpu.sync_copy(params_hbm, params_v)
    iv = params_v[pl.ds(0, _L)]
    ap = params_v[pl.ds(_L, _L)]
    am = params_v[pl.ds(2 * _L, _L)]
    iota = lax.iota(jnp.int32, _L)

    for g in range(groups):
        rows = wid * rows_per_w + g * _L
        pltpu.sync_copy(ch_hbm.at[pl.ds(rows, _L), :],
                        ch_v.at[:, pl.ds(0, T)])
        pltpu.sync_copy(rw_hbm.at[pl.ds(rows, _L), :],
                        rw_v.at[:, pl.ds(0, T)])

        for l in range(_L):
            for j in range(2):
                stage_v[l, 0, pl.ds(j * _L, _L)] = iv

        def step(t, carry):
            t_vec = jnp.full((_L,), t, jnp.int32)
            ch = plsc.load_gather(ch_v, [iota, t_vec])
            rw = plsc.load_gather(rw_v, [iota, t_vec])
            chosen = plsc.load_gather(stage_v, [iota, t_vec, ch])
            pe = rw - chosen
            pe = jnp.where(rw != rw, 0.0, pe)
            coef = jnp.where(pe >= 0, ap, am)
            upd = chosen + coef * pe
            for l in range(_L):
                for j in range(2):
                    stage_v[l, t + 1, pl.ds(j * _L, _L)] = (
                        stage_v[l, t, pl.ds(j * _L, _L)])
            plsc.store_scatter(stage_v, [iota, t_vec + 1, ch], upd)
            return carry

        lax.fori_loop(0, T - 1, step, 0)

        pltpu.async_copy(stage_v, out_hbm.at[pl.ds(rows, _L)], sem).wait()


def kernel(choices, rewards, alpha_plus, alpha_minus, initial_values):
    N, T = choices.shape
    Tp = T + 1
    iv = 100.0 * jnp.tanh(initial_values)
    ap = jax.nn.sigmoid(alpha_plus)
    am = jax.nn.sigmoid(alpha_minus)
    params = jnp.concatenate([
        jnp.full((_L,), iv, jnp.float32),
        jnp.full((_L,), ap, jnp.float32),
        jnp.full((_L,), am, jnp.float32),
    ])

    mesh = plsc.VectorSubcoreMesh(core_axis_name="c", subcore_axis_name="s")
    run = pl.kernel(
        functools.partial(_sc_body, T, Tp),
        out_type=jax.ShapeDtypeStruct((N, T, _K), jnp.float32),
        mesh=mesh,
        scratch_types=[
            pltpu.VMEM((3 * _L,), jnp.float32),
            pltpu.VMEM((_L, Tp), jnp.int32),
            pltpu.VMEM((_L, Tp), jnp.float32),
            pltpu.VMEM((_L, T, _K), jnp.float32),
            pltpu.SemaphoreType.DMA,
        ],
        compiler_params=pltpu.CompilerParams(
            use_tc_tiling_on_sc=False, needs_layout_passes=False),
    )
    return run(params, choices, rewards)


# speedup vs baseline: 15.5689x; 1.7960x over previous
import functools

import jax
import jax.numpy as jnp
from jax import lax
from jax.experimental import pallas as pl
from jax.experimental.pallas import tpu as pltpu
from jax.experimental.pallas import tpu_sc as plsc

_K = 32
_L = 16
_NW = 32


def _sc_body(T, params_hbm, ch_hbm, rw_hbm, out_hbm,
             params_v, ch_v, rw_v, stage_v, sem):
    wid = lax.axis_index("s") * 2 + lax.axis_index("c")
    rows_per_w = ch_hbm.shape[0] // _NW
    groups = rows_per_w // _L

    pltpu.sync_copy(params_hbm, params_v)
    iv = params_v[pl.ds(0, _L)]
    ap = params_v[pl.ds(_L, _L)]
    am = params_v[pl.ds(2 * _L, _L)]
    iota = lax.iota(jnp.int32, _L)

    for g in range(groups):
        rows = wid * rows_per_w + g * _L
        pltpu.sync_copy(ch_hbm.at[pl.ds(rows, _L), :], ch_v)
        pltpu.sync_copy(rw_hbm.at[pl.ds(rows, _L), :], rw_v)

        for l in range(_L):
            for j in range(2):
                stage_v[l, pl.ds(j * _L, _L)] = iv

        def step(t, carry):
            t_vec = jnp.full((_L,), t, jnp.int32)
            ch = plsc.load_gather(ch_v, [iota, t_vec])
            rw = plsc.load_gather(rw_v, [iota, t_vec])
            col = t * _K
            kpos = t_vec * _K + ch
            chosen = plsc.load_gather(stage_v, [iota, kpos])
            pe = rw - chosen
            pe = jnp.where(rw != rw, 0.0, pe)
            coef = jnp.where(pe >= 0, ap, am)
            upd = chosen + coef * pe
            for l in range(_L):
                for j in range(2):
                    stage_v[l, pl.ds(col + _K + j * _L, _L)] = (
                        stage_v[l, pl.ds(col + j * _L, _L)])
            plsc.store_scatter(stage_v, [iota, kpos + _K], upd)
            return carry

        lax.fori_loop(0, T - 1, step, 0)

        pltpu.async_copy(stage_v, out_hbm.at[pl.ds(rows, _L)], sem).wait()


def kernel(choices, rewards, alpha_plus, alpha_minus, initial_values):
    N, T = choices.shape
    iv = 100.0 * jnp.tanh(initial_values)
    ap = jax.nn.sigmoid(alpha_plus)
    am = jax.nn.sigmoid(alpha_minus)
    params = jnp.concatenate([
        jnp.full((_L,), iv, jnp.float32),
        jnp.full((_L,), ap, jnp.float32),
        jnp.full((_L,), am, jnp.float32),
    ])

    mesh = plsc.VectorSubcoreMesh(core_axis_name="c", subcore_axis_name="s")
    run = pl.kernel(
        functools.partial(_sc_body, T),
        out_type=jax.ShapeDtypeStruct((N, T * _K), jnp.float32),
        mesh=mesh,
        scratch_types=[
            pltpu.VMEM((3 * _L,), jnp.float32),
            pltpu.VMEM((_L, T), jnp.int32),
            pltpu.VMEM((_L, T), jnp.float32),
            pltpu.VMEM((_L, T * _K), jnp.float32),
            pltpu.SemaphoreType.DMA,
        ],
        compiler_params=pltpu.CompilerParams(
            use_tc_tiling_on_sc=False, needs_layout_passes=False),
    )
    return run(params, choices, rewards).reshape(N, T, _K)
